# baseline (speedup 1.0000x reference)
import functools

import jax
import jax.numpy as jnp
from jax import lax
from jax.experimental import pallas as pl
from jax.experimental.pallas import tpu as pltpu
from jax.experimental.pallas import tpu_sc as plsc

_B = 16384
_F = 32

_NC, _NS = 2, 16
_NW = _NC * _NS
_BPW = _B // _NW


_NROWS = 1000000


def _sc_gather_body(uid_ref, iid_ref, ugw_ref, igw_ref, umw_ref, imw_ref,
                    gmf_ref, um_ref, im_ref,
                    uidx_v, iidx_v, ug_v, ig_v, um_v, im_v,
                    sems):
    wid = lax.axis_index("s") * _NC + lax.axis_index("c")
    base = wid * _BPW
    pltpu.sync_copy(uid_ref.at[wid], uidx_v)
    pltpu.sync_copy(iid_ref.at[wid], iidx_v)

    def col(c, carry):
        for j in range(_BPW // 128):
            dst = pl.ds(j * 128, 128)
            pltpu.async_copy(ugw_ref.at[c].at[uidx_v.at[j]], ug_v.at[c, dst],
                             sems.at[0])
            pltpu.async_copy(igw_ref.at[c].at[iidx_v.at[j]], ig_v.at[c, dst],
                             sems.at[1])
            pltpu.async_copy(umw_ref.at[c].at[uidx_v.at[j]], um_v.at[c, dst],
                             sems.at[2])
            pltpu.async_copy(imw_ref.at[c].at[iidx_v.at[j]], im_v.at[c, dst],
                             sems.at[3])
        return carry

    lax.fori_loop(0, _F, col, 0)
    bufs = (ug_v, ig_v, um_v, im_v)
    for t in range(4):
        pltpu.make_async_copy(um_ref.at[:, pl.ds(0, _BPW)], bufs[t],
                              sems.at[t]).wait()

    def prod(c, carry):
        for h in range(_BPW // 16):
            sl = pl.ds(h * 16, 16)
            ug_v[c, sl] = ug_v[c, sl] * ig_v[c, sl]
        return carry

    lax.fori_loop(0, _F, prod, 0)
    cols = pl.ds(base, _BPW)
    pltpu.sync_copy(ug_v, gmf_ref.at[:, cols])
    pltpu.sync_copy(um_v, um_ref.at[:, cols])
    pltpu.sync_copy(im_v, im_ref.at[:, cols])


@functools.cache
def _build_sc_gather():
  return pl.kernel(
    _sc_gather_body,
    out_type=(
        jax.ShapeDtypeStruct((_F, _B), jnp.float32),
        jax.ShapeDtypeStruct((_F, _B), jnp.float32),
        jax.ShapeDtypeStruct((_F, _B), jnp.float32),
    ),
    mesh=plsc.VectorSubcoreMesh(core_axis_name="c", subcore_axis_name="s"),
    compiler_params=pltpu.CompilerParams(use_tc_tiling_on_sc=False),
    scratch_types=[
        pltpu.VMEM((_BPW // 128, 128), jnp.int32),
        pltpu.VMEM((_BPW // 128, 128), jnp.int32),
        pltpu.VMEM((_F, _BPW), jnp.float32),
        pltpu.VMEM((_F, _BPW), jnp.float32),
        pltpu.VMEM((_F, _BPW), jnp.float32),
        pltpu.VMEM((_F, _BPW), jnp.float32),
        pltpu.SemaphoreType.DMA((4,)),
    ],
  )


def _tc_mlp_body(gmf_ref, um_ref, im_ref, w1u_ref, w1i_ref, b1_ref,
                 w2_ref, b2_ref, w3_ref, b3_ref, wog_ref, woh_ref, bo_ref,
                 out_ref):
    f32 = jnp.float32
    h = jnp.dot(w1u_ref[:], um_ref[:], preferred_element_type=f32)
    h = h + jnp.dot(w1i_ref[:], im_ref[:], preferred_element_type=f32)
    h = jnp.maximum(h + b1_ref[:], 0.0)
    h = jnp.maximum(
        jnp.dot(w2_ref[:], h, preferred_element_type=f32) + b2_ref[:], 0.0)
    h = jnp.maximum(
        jnp.dot(w3_ref[:], h, preferred_element_type=f32) + b3_ref[:], 0.0)
    logit = jnp.dot(wog_ref[:], gmf_ref[:], preferred_element_type=f32)
    logit = logit + jnp.dot(woh_ref[:], h, preferred_element_type=f32)
    logit = logit + bo_ref[:]
    out_ref[:] = jax.nn.sigmoid(logit)


_TC_BLOCK = 2048
_TC_GRID = _B // _TC_BLOCK


def _full(shape):
    return pl.BlockSpec(shape, lambda i: (0,) * len(shape))


_tc_mlp = pl.pallas_call(
    _tc_mlp_body,
    grid=(_TC_GRID,),
    in_specs=[
        pl.BlockSpec((_F, _TC_BLOCK), lambda i: (0, i)),
        pl.BlockSpec((_F, _TC_BLOCK), lambda i: (0, i)),
        pl.BlockSpec((_F, _TC_BLOCK), lambda i: (0, i)),
        _full((64, _F)), _full((64, _F)), _full((64, 1)),
        _full((32, 64)), _full((32, 1)),
        _full((16, 32)), _full((16, 1)),
        _full((1, _F)), _full((1, 16)), _full((1, 1)),
    ],
    out_specs=pl.BlockSpec((1, _TC_BLOCK), lambda i: (0, i)),
    out_shape=jax.ShapeDtypeStruct((1, _B), jnp.float32),
    compiler_params=pltpu.CompilerParams(
        dimension_semantics=("arbitrary",)),
)


@jax.jit
def kernel(user_id, item_id, user_gmf_w, item_gmf_w, user_mlp_w, item_mlp_w,
           W1, b1, W2, b2, W3, b3, Wo, bo):
    uid2 = user_id.astype(jnp.int32).reshape(_NW, _BPW // 128, 128)
    iid2 = item_id.astype(jnp.int32).reshape(_NW, _BPW // 128, 128)
    gmf, um, im = _build_sc_gather()(uid2, iid2, user_gmf_w.T, item_gmf_w.T,
                                     user_mlp_w.T, item_mlp_w.T)
    out = _tc_mlp(gmf, um, im,
                  W1[:_F].T, W1[_F:].T, b1.reshape(64, 1),
                  W2.T, b2.reshape(32, 1),
                  W3.T, b3.reshape(16, 1),
                  Wo[:_F].T, Wo[_F:].T, bo.reshape(1, 1))
    return jnp.squeeze(out, axis=0)


# speedup vs baseline: 4.0449x; 4.0449x over previous
import functools

import jax
import jax.numpy as jnp
from jax import lax
from jax.experimental import pallas as pl
from jax.experimental.pallas import tpu as pltpu
from jax.experimental.pallas import tpu_sc as plsc

_B = 16384
_F = 32

_NC, _NS = 2, 16
_NW = _NC * _NS
_BPW = _B // _NW


_NFULL = 1000000 // 128
_TAIL = 1000000 - _NFULL * 128


def _sc_relayout_body(ugt_ref, igt_ref, umt_ref, imt_ref,
                      tug_ref, tig_ref, tum_ref, tim_ref,
                      ug_ref, ig_ref, um_ref, im_ref,
                      ibufs, obufs, sem_i, sem_o):
    wid = lax.axis_index("s") * _NC + lax.axis_index("c")
    srcs = (ugt_ref, igt_ref, umt_ref, imt_ref)
    tails = (tug_ref, tig_ref, tum_ref, tim_ref)
    dsts = (ug_ref, ig_ref, um_ref, im_ref)
    lanes = lax.iota(jnp.int32, 16)
    niter = _NFULL // _NW + 1

    @pl.when(wid == 0)
    def _tail():
        for t in range(4):
            pltpu.sync_copy(tails[t], obufs.at[t, pl.ds(0, _TAIL)])
            pltpu.sync_copy(obufs.at[t, pl.ds(0, _TAIL)],
                            dsts[t].at[pl.ds(_NFULL * 128, _TAIL)])

    def r_of(i):
        return lax.rem(i * _NW + wid, _NFULL) * 128

    for t in range(4):
        pltpu.async_copy(srcs[t].at[:, pl.ds(r_of(0), 128)],
                         ibufs.at[0, t], sem_i.at[t])
        pltpu.async_copy(dsts[t].at[pl.ds(0, 128)], obufs.at[t], sem_o.at[t])

    def chunk(i, carry):
        b = lax.rem(i, 2)
        for t in range(4):
            pltpu.make_async_copy(srcs[t].at[:, pl.ds(0, 128)],
                                  ibufs.at[b, t], sem_i.at[t]).wait()
        for t in range(4):
            pltpu.async_copy(srcs[t].at[:, pl.ds(r_of(i + 1), 128)],
                             ibufs.at[1 - b, t], sem_i.at[t])
        r0 = r_of(i)
        for t in range(4):
            pltpu.make_async_copy(dsts[t].at[pl.ds(0, 128)], obufs.at[t],
                                  sem_o.at[t]).wait()

            def col(c, carry2):
                for g in range(8):
                    v = ibufs[b, t, c, pl.ds(g * 16, 16)]
                    plsc.store_scatter(
                        obufs.at[t], [lanes + g * 16,
                                      jnp.broadcast_to(c, (16,))], v)
                return carry2

            lax.fori_loop(0, _F, col, 0)
            pltpu.async_copy(obufs.at[t], dsts[t].at[pl.ds(r0, 128)],
                             sem_o.at[t])
        return carry

    lax.fori_loop(0, niter, chunk, 0)
    for t in range(4):
        pltpu.make_async_copy(srcs[t].at[:, pl.ds(0, 128)],
                              ibufs.at[lax.rem(niter, 2), t],
                              sem_i.at[t]).wait()
        pltpu.make_async_copy(dsts[t].at[pl.ds(0, 128)], obufs.at[t],
                              sem_o.at[t]).wait()


@functools.cache
def _build_sc_relayout():
  return pl.kernel(
    _sc_relayout_body,
    out_type=(
        jax.ShapeDtypeStruct((1000000, _F), jnp.float32),
        jax.ShapeDtypeStruct((1000000, _F), jnp.float32),
        jax.ShapeDtypeStruct((1000000, _F), jnp.float32),
        jax.ShapeDtypeStruct((1000000, _F), jnp.float32),
    ),
    mesh=plsc.VectorSubcoreMesh(core_axis_name="c", subcore_axis_name="s"),
    compiler_params=pltpu.CompilerParams(needs_layout_passes=False),
    scratch_types=[
        pltpu.VMEM((2, 4, _F, 128), jnp.float32),
        pltpu.VMEM((4, 128, _F), jnp.float32),
        pltpu.SemaphoreType.DMA((4,)),
        pltpu.SemaphoreType.DMA((4,)),
    ],
  )


def _sc_gather_body(uid_ref, iid_ref, ugw_ref, igw_ref, umw_ref, imw_ref,
                    gmf_ref, um_ref, im_ref,
                    uidx_v, iidx_v,
                    ug_v, ig_v, um_v, im_v, sems):
    wid = lax.axis_index("s") * _NC + lax.axis_index("c")
    base = wid * _BPW
    pltpu.sync_copy(uid_ref.at[wid], uidx_v)
    pltpu.sync_copy(iid_ref.at[wid], iidx_v)

    quarter = _BPW // 4
    bufs = (ug_v, ig_v, um_v, im_v)
    for qq in range(4):
        off = qq * quarter

        def grp(g, carry):
            u16 = uidx_v[pl.ds(off + g * 16, 16)]
            i16 = iidx_v[pl.ds(off + g * 16, 16)]
            for j in range(16):
                ru = u16[j]
                ri = i16[j]
                dst = pl.ds(g * 16 + j, 1)
                pltpu.async_copy(ugw_ref.at[pl.ds(ru, 1)], ug_v.at[dst],
                                 sems.at[0])
                pltpu.async_copy(igw_ref.at[pl.ds(ri, 1)], ig_v.at[dst],
                                 sems.at[1])
                pltpu.async_copy(umw_ref.at[pl.ds(ru, 1)], um_v.at[dst],
                                 sems.at[2])
                pltpu.async_copy(imw_ref.at[pl.ds(ri, 1)], im_v.at[dst],
                                 sems.at[3])
            return carry

        lax.fori_loop(0, quarter // 16, grp, 0)
        for t in range(4):
            pltpu.make_async_copy(um_ref.at[pl.ds(0, quarter)], bufs[t],
                                  sems.at[t]).wait()

        def prod(r, carry):
            for h in range(_F // 16):
                sl = pl.ds(h * 16, 16)
                ug_v[r, sl] = ug_v[r, sl] * ig_v[r, sl]
            return carry

        lax.fori_loop(0, quarter, prod, 0)
        rows = pl.ds(base + off, quarter)
        pltpu.sync_copy(ug_v, gmf_ref.at[rows])
        pltpu.sync_copy(um_v, um_ref.at[rows])
        pltpu.sync_copy(im_v, im_ref.at[rows])


@functools.cache
def _build_sc_gather():
  return pl.kernel(
    _sc_gather_body,
    out_type=(
        jax.ShapeDtypeStruct((_B, _F), jnp.float32),
        jax.ShapeDtypeStruct((_B, _F), jnp.float32),
        jax.ShapeDtypeStruct((_B, _F), jnp.float32),
    ),
    mesh=plsc.VectorSubcoreMesh(core_axis_name="c", subcore_axis_name="s"),
    scratch_types=[
        pltpu.VMEM((_BPW,), jnp.int32),
        pltpu.VMEM((_BPW,), jnp.int32),
        pltpu.VMEM((_BPW // 4, _F), jnp.float32),
        pltpu.VMEM((_BPW // 4, _F), jnp.float32),
        pltpu.VMEM((_BPW // 4, _F), jnp.float32),
        pltpu.VMEM((_BPW // 4, _F), jnp.float32),
        pltpu.SemaphoreType.DMA((4,)),
    ],
  )


def _tc_mlp_body(gmf_ref, um_ref, im_ref, w1u_ref, w1i_ref, b1_ref,
                 w2_ref, b2_ref, w3_ref, b3_ref, wog_ref, woh_ref, bo_ref,
                 out_ref):
    f32 = jnp.float32
    h = jnp.dot(um_ref[:], w1u_ref[:], preferred_element_type=f32)
    h = h + jnp.dot(im_ref[:], w1i_ref[:], preferred_element_type=f32)
    h = jnp.maximum(h + b1_ref[:], 0.0)
    h = jnp.maximum(
        jnp.dot(h, w2_ref[:], preferred_element_type=f32) + b2_ref[:], 0.0)
    h = jnp.maximum(
        jnp.dot(h, w3_ref[:], preferred_element_type=f32) + b3_ref[:], 0.0)
    logit = jnp.dot(gmf_ref[:], wog_ref[:], preferred_element_type=f32)
    logit = logit + jnp.dot(h, woh_ref[:], preferred_element_type=f32)
    logit = logit + bo_ref[:]
    out_ref[:] = jax.nn.sigmoid(logit)


_TC_BLOCK = 2048
_TC_GRID = _B // _TC_BLOCK


def _full(shape):
    return pl.BlockSpec(shape, lambda i: (0,) * len(shape))


_tc_mlp = pl.pallas_call(
    _tc_mlp_body,
    grid=(_TC_GRID,),
    in_specs=[
        pl.BlockSpec((_TC_BLOCK, _F), lambda i: (i, 0)),
        pl.BlockSpec((_TC_BLOCK, _F), lambda i: (i, 0)),
        pl.BlockSpec((_TC_BLOCK, _F), lambda i: (i, 0)),
        _full((_F, 64)), _full((_F, 64)), _full((1, 64)),
        _full((64, 32)), _full((1, 32)),
        _full((32, 16)), _full((1, 16)),
        _full((_F, 1)), _full((16, 1)), _full((1, 1)),
    ],
    out_specs=pl.BlockSpec((_TC_BLOCK, 1), lambda i: (i, 0)),
    out_shape=jax.ShapeDtypeStruct((_B, 1), jnp.float32),
    compiler_params=pltpu.CompilerParams(
        dimension_semantics=("arbitrary",)),
)


@jax.jit
def kernel(user_id, item_id, user_gmf_w, item_gmf_w, user_mlp_w, item_mlp_w,
           W1, b1, W2, b2, W3, b3, Wo, bo):
    uid2 = user_id.astype(jnp.int32).reshape(_NW, _BPW)
    iid2 = item_id.astype(jnp.int32).reshape(_NW, _BPW)
    t0 = _NFULL * 128
    ug_rm, ig_rm, um_rm, im_rm = _build_sc_relayout()(
        user_gmf_w.T, item_gmf_w.T, user_mlp_w.T, item_mlp_w.T,
        user_gmf_w[t0:], item_gmf_w[t0:], user_mlp_w[t0:], item_mlp_w[t0:])
    gmf, um, im = _build_sc_gather()(uid2, iid2, ug_rm, ig_rm,
                                     um_rm, im_rm)
    out = _tc_mlp(gmf, um, im,
                  W1[:_F], W1[_F:], b1.reshape(1, 64),
                  W2, b2.reshape(1, 32),
                  W3, b3.reshape(1, 16),
                  Wo[:_F], Wo[_F:], bo.reshape(1, 1))
    return jnp.squeeze(out, axis=-1)


# speedup vs baseline: 8.4641x; 2.0926x over previous
import functools

import jax
import jax.numpy as jnp
from jax import lax
from jax.experimental import pallas as pl
from jax.experimental.pallas import tpu as pltpu
from jax.experimental.pallas import tpu_sc as plsc

_B = 16384
_F = 32

_NC, _NS = 2, 16
_NW = _NC * _NS
_BPW = _B // _NW


def _sc_gather_body(uid_ref, iid_ref, ugw_ref, igw_ref, umw_ref, imw_ref,
                    gmf_ref, um_ref, im_ref,
                    uidx_v, iidx_v,
                    ug_v, ig_v, um_v, im_v, sems):
    wid = lax.axis_index("s") * _NC + lax.axis_index("c")
    base = wid * _BPW
    pltpu.sync_copy(uid_ref.at[wid], uidx_v)
    pltpu.sync_copy(iid_ref.at[wid], iidx_v)

    quarter = _BPW // 4
    bufs = (ug_v, ig_v, um_v, im_v)
    for qq in range(4):
        off = qq * quarter

        def grp(g, carry):
            u16 = uidx_v[pl.ds(off + g * 16, 16)]
            i16 = iidx_v[pl.ds(off + g * 16, 16)]
            for j in range(16):
                ru = u16[j]
                ri = i16[j]
                dst = pl.ds(g * 16 + j, 1)
                pltpu.async_copy(ugw_ref.at[pl.ds(ru, 1)], ug_v.at[dst],
                                 sems.at[0])
                pltpu.async_copy(igw_ref.at[pl.ds(ri, 1)], ig_v.at[dst],
                                 sems.at[1])
                pltpu.async_copy(umw_ref.at[pl.ds(ru, 1)], um_v.at[dst],
                                 sems.at[2])
                pltpu.async_copy(imw_ref.at[pl.ds(ri, 1)], im_v.at[dst],
                                 sems.at[3])
            return carry

        lax.fori_loop(0, quarter // 16, grp, 0)
        for t in range(4):
            pltpu.make_async_copy(um_ref.at[pl.ds(0, quarter)], bufs[t],
                                  sems.at[t]).wait()

        def prod(r, carry):
            for h in range(_F // 16):
                sl = pl.ds(h * 16, 16)
                ug_v[r, sl] = ug_v[r, sl] * ig_v[r, sl]
            return carry

        lax.fori_loop(0, quarter, prod, 0)
        rows = pl.ds(base + off, quarter)
        pltpu.sync_copy(ug_v, gmf_ref.at[rows])
        pltpu.sync_copy(um_v, um_ref.at[rows])
        pltpu.sync_copy(im_v, im_ref.at[rows])


@functools.cache
def _build_sc_gather():
  return pl.kernel(
    _sc_gather_body,
    out_type=(
        jax.ShapeDtypeStruct((_B, _F), jnp.float32),
        jax.ShapeDtypeStruct((_B, _F), jnp.float32),
        jax.ShapeDtypeStruct((_B, _F), jnp.float32),
    ),
    mesh=plsc.VectorSubcoreMesh(core_axis_name="c", subcore_axis_name="s"),
    scratch_types=[
        pltpu.VMEM((_BPW,), jnp.int32),
        pltpu.VMEM((_BPW,), jnp.int32),
        pltpu.VMEM((_BPW // 4, _F), jnp.float32),
        pltpu.VMEM((_BPW // 4, _F), jnp.float32),
        pltpu.VMEM((_BPW // 4, _F), jnp.float32),
        pltpu.VMEM((_BPW // 4, _F), jnp.float32),
        pltpu.SemaphoreType.DMA((4,)),
    ],
  )


def _tc_mlp_body(gmf_ref, um_ref, im_ref, w1u_ref, w1i_ref, b1_ref,
                 w2_ref, b2_ref, w3_ref, b3_ref, wog_ref, woh_ref, bo_ref,
                 out_ref):
    f32 = jnp.float32
    h = jnp.dot(um_ref[:], w1u_ref[:], preferred_element_type=f32)
    h = h + jnp.dot(im_ref[:], w1i_ref[:], preferred_element_type=f32)
    h = jnp.maximum(h + b1_ref[:], 0.0)
    h = jnp.maximum(
        jnp.dot(h, w2_ref[:], preferred_element_type=f32) + b2_ref[:], 0.0)
    h = jnp.maximum(
        jnp.dot(h, w3_ref[:], preferred_element_type=f32) + b3_ref[:], 0.0)
    logit = jnp.dot(gmf_ref[:], wog_ref[:], preferred_element_type=f32)
    logit = logit + jnp.dot(h, woh_ref[:], preferred_element_type=f32)
    logit = logit + bo_ref[:]
    out_ref[:] = jax.nn.sigmoid(logit)


_TC_BLOCK = 2048
_TC_GRID = _B // _TC_BLOCK


def _full(shape):
    return pl.BlockSpec(shape, lambda i: (0,) * len(shape))


_tc_mlp = pl.pallas_call(
    _tc_mlp_body,
    grid=(_TC_GRID,),
    in_specs=[
        pl.BlockSpec((_TC_BLOCK, _F), lambda i: (i, 0)),
        pl.BlockSpec((_TC_BLOCK, _F), lambda i: (i, 0)),
        pl.BlockSpec((_TC_BLOCK, _F), lambda i: (i, 0)),
        _full((_F, 64)), _full((_F, 64)), _full((1, 64)),
        _full((64, 32)), _full((1, 32)),
        _full((32, 16)), _full((1, 16)),
        _full((_F, 1)), _full((16, 1)), _full((1, 1)),
    ],
    out_specs=pl.BlockSpec((_TC_BLOCK, 1), lambda i: (i, 0)),
    out_shape=jax.ShapeDtypeStruct((_B, 1), jnp.float32),
    compiler_params=pltpu.CompilerParams(
        dimension_semantics=("arbitrary",)),
)


@jax.jit
def kernel(user_id, item_id, user_gmf_w, item_gmf_w, user_mlp_w, item_mlp_w,
           W1, b1, W2, b2, W3, b3, Wo, bo):
    uid2 = user_id.astype(jnp.int32).reshape(_NW, _BPW)
    iid2 = item_id.astype(jnp.int32).reshape(_NW, _BPW)
    gmf, um, im = _build_sc_gather()(uid2, iid2, user_gmf_w, item_gmf_w,
                                     user_mlp_w, item_mlp_w)
    out = _tc_mlp(gmf, um, im,
                  W1[:_F], W1[_F:], b1.reshape(1, 64),
                  W2, b2.reshape(1, 32),
                  W3, b3.reshape(1, 16),
                  Wo[:_F], Wo[_F:], bo.reshape(1, 1))
    return jnp.squeeze(out, axis=-1)
